# baseline (speedup 1.0000x reference)
import functools

import jax
import jax.numpy as jnp
from jax import lax
from jax.experimental import pallas as pl
from jax.experimental.pallas import tpu as pltpu
from jax.experimental.pallas import tpu_sc as plsc

N_NODES = 100000
H_DIM = 128
NUM_RELS = 100
N_TRIPLETS = 500000

NC = 2
NS = 16
L = 16
NW = NC * NS

C = 128


def _score_body(n_chunks, b_per_w,
                emb_hbm, wrel_hbm, src_hbm, rel_hbm, dst_hbm, out_hbm,
                wrel_v, sidx_v, didx_v, ridx_v, srows_v, orows_v, scores_v,
                sem):
    wid = lax.axis_index("s") * NC + lax.axis_index("c")
    base_w = wid * b_per_w
    pltpu.sync_copy(wrel_hbm, wrel_v)
    lane = lax.iota(jnp.int32, L)

    def chunk_body(i, carry):
        base = base_w + i * C
        pltpu.sync_copy(src_hbm.at[pl.ds(base, C)], sidx_v)
        pltpu.sync_copy(dst_hbm.at[pl.ds(base, C)], didx_v)
        pltpu.sync_copy(rel_hbm.at[pl.ds(base, C)], ridx_v)
        cp_s = pltpu.async_copy(emb_hbm.at[sidx_v], srows_v, sem)
        cp_o = pltpu.async_copy(emb_hbm.at[didx_v], orows_v, sem)
        cp_s.wait()
        cp_o.wait()

        def group_body(g, gcarry):
            rows = lane + g * L
            rvec = ridx_v[pl.ds(g * L, L)]
            acc = jnp.zeros((L,), jnp.float32)
            for f in range(H_DIM):
                col = jnp.full((L,), f, jnp.int32)
                sv = plsc.load_gather(srows_v, [rows, col])
                ov = plsc.load_gather(orows_v, [rows, col])
                rv = plsc.load_gather(wrel_v, [rvec, col])
                acc = acc + sv * ov * rv
            scores_v[pl.ds(g * L, L)] = acc
            return gcarry

        lax.fori_loop(0, C // L, group_body, 0)
        pltpu.sync_copy(scores_v, out_hbm.at[pl.ds(base, C)])
        return carry

    lax.fori_loop(0, n_chunks, chunk_body, 0)


def kernel(embedding, w_relation, src, rel, dst):
    n = src.shape[0]
    step = NW * C
    n_pad = ((n + step - 1) // step) * step
    pad = n_pad - n
    if pad:
        zpad = jnp.zeros((pad,), src.dtype)
        src = jnp.concatenate([src, zpad])
        rel = jnp.concatenate([rel, zpad])
        dst = jnp.concatenate([dst, zpad])
    b_per_w = n_pad // NW
    n_chunks = b_per_w // C

    mesh = plsc.VectorSubcoreMesh(core_axis_name="c", subcore_axis_name="s")
    body = functools.partial(_score_body, n_chunks, b_per_w)
    score = pl.kernel(
        body,
        out_type=jax.ShapeDtypeStruct((n_pad,), jnp.float32),
        mesh=mesh,
        compiler_params=pltpu.CompilerParams(needs_layout_passes=False),
        scratch_types=[
            pltpu.VMEM((NUM_RELS, H_DIM), jnp.float32),
            pltpu.VMEM((C,), jnp.int32),
            pltpu.VMEM((C,), jnp.int32),
            pltpu.VMEM((C,), jnp.int32),
            pltpu.VMEM((C, H_DIM), jnp.float32),
            pltpu.VMEM((C, H_DIM), jnp.float32),
            pltpu.VMEM((C,), jnp.float32),
            pltpu.SemaphoreType.DMA,
        ],
    )(embedding, w_relation, src, rel, dst)
    return score[:n]


# speedup vs baseline: 1.1739x; 1.1739x over previous
import functools

import jax
import jax.numpy as jnp
from jax import lax
from jax.experimental import pallas as pl
from jax.experimental.pallas import tpu as pltpu
from jax.experimental.pallas import tpu_sc as plsc

N_NODES = 100000
H_DIM = 128
NUM_RELS = 100
N_TRIPLETS = 500000

NC = 2
NS = 16
L = 16
NW = NC * NS

C = 128
FB = 16


def _score_body(n_chunks, b_per_w,
                emb_hbm, wrel_hbm, src_hbm, rel_hbm, dst_hbm, out_hbm,
                wrel_v, sidx_v, didx_v, ridx_v, srows_v, orows_v, scores_v,
                sem_idx, sem_rows, sem_out):
    wid = lax.axis_index("s") * NC + lax.axis_index("c")
    base_w = wid * b_per_w
    pltpu.sync_copy(wrel_hbm, wrel_v)
    lane = lax.iota(jnp.int32, L)

    def start_idx(chunk, slot):
        base = base_w + chunk * C
        pltpu.async_copy(src_hbm.at[pl.ds(base, C)], sidx_v[slot], sem_idx)
        pltpu.async_copy(dst_hbm.at[pl.ds(base, C)], didx_v[slot], sem_idx)
        pltpu.async_copy(rel_hbm.at[pl.ds(base, C)], ridx_v[slot], sem_idx)

    def wait_idx(slot):
        pltpu.make_async_copy(src_hbm.at[pl.ds(0, C)], sidx_v[slot], sem_idx).wait()
        pltpu.make_async_copy(dst_hbm.at[pl.ds(0, C)], didx_v[slot], sem_idx).wait()
        pltpu.make_async_copy(rel_hbm.at[pl.ds(0, C)], ridx_v[slot], sem_idx).wait()

    def start_rows(slot):
        pltpu.async_copy(emb_hbm.at[sidx_v[slot]], srows_v[slot], sem_rows[slot])
        pltpu.async_copy(emb_hbm.at[didx_v[slot]], orows_v[slot], sem_rows[slot])

    def wait_rows(slot):
        pltpu.make_async_copy(emb_hbm.at[sidx_v[slot]], srows_v[slot], sem_rows[slot]).wait()
        pltpu.make_async_copy(emb_hbm.at[didx_v[slot]], orows_v[slot], sem_rows[slot]).wait()

    def start_out(chunk, slot):
        pltpu.async_copy(scores_v[slot], out_hbm.at[pl.ds(base_w + chunk * C, C)],
                         sem_out[slot])

    def wait_out(slot):
        pltpu.make_async_copy(scores_v[slot], out_hbm.at[pl.ds(0, C)], sem_out[slot]).wait()

    def compute(slot):
        def group_body(g, gcarry):
            rows = lane + g * L
            rvec = ridx_v[slot][pl.ds(g * L, L)]

            def fblk_body(fb, acc):
                fbase = fb * FB
                for ff in range(FB):
                    col = jnp.full((L,), ff, jnp.int32) + fbase
                    sv = plsc.load_gather(srows_v[slot], [rows, col])
                    ov = plsc.load_gather(orows_v[slot], [rows, col])
                    rv = plsc.load_gather(wrel_v, [rvec, col])
                    acc = acc + sv * ov * rv
                return acc

            acc = lax.fori_loop(0, H_DIM // FB, fblk_body, jnp.zeros((L,), jnp.float32))
            scores_v[slot][pl.ds(g * L, L)] = acc
            return gcarry

        lax.fori_loop(0, C // L, group_body, 0)

    start_idx(0, 0)
    wait_idx(0)
    start_rows(0)
    start_idx(1, 1)
    for i in (0, 1):
        b, nxt = i % 2, (i + 1) % 2
        wait_idx(nxt)
        start_rows(nxt)
        wait_rows(b)
        compute(b)
        start_out(i, b)
        start_idx(i + 2, b)

    def block_body(g, carry):
        for b in (0, 1):
            i = 2 + 2 * g + b
            nxt = 1 - b
            wait_idx(nxt)
            start_rows(nxt)
            wait_rows(b)
            wait_out(b)
            compute(b)
            start_out(i, b)
            start_idx(i + 2, b)
        return carry

    lax.fori_loop(0, (n_chunks - 4) // 2, block_body, 0)

    i = n_chunks - 2
    wait_idx(1)
    start_rows(1)
    wait_rows(0)
    wait_out(0)
    compute(0)
    start_out(i, 0)
    wait_rows(1)
    wait_out(1)
    compute(1)
    start_out(i + 1, 1)
    wait_out(0)
    wait_out(1)


def kernel(embedding, w_relation, src, rel, dst):
    n = src.shape[0]
    step = NW * C * 2
    n_pad = ((n + step - 1) // step) * step
    pad = n_pad - n
    if pad:
        zpad = jnp.zeros((pad,), src.dtype)
        src = jnp.concatenate([src, zpad])
        rel = jnp.concatenate([rel, zpad])
        dst = jnp.concatenate([dst, zpad])
    b_per_w = n_pad // NW
    n_chunks = b_per_w // C
    assert n_chunks >= 6 and n_chunks % 2 == 0

    mesh = plsc.VectorSubcoreMesh(core_axis_name="c", subcore_axis_name="s")
    body = functools.partial(_score_body, n_chunks, b_per_w)
    score = pl.kernel(
        body,
        out_type=jax.ShapeDtypeStruct((n_pad,), jnp.float32),
        mesh=mesh,
        compiler_params=pltpu.CompilerParams(needs_layout_passes=False),
        scratch_types=[
            pltpu.VMEM((NUM_RELS, H_DIM), jnp.float32),
            [pltpu.VMEM((C,), jnp.int32) for _ in range(2)],
            [pltpu.VMEM((C,), jnp.int32) for _ in range(2)],
            [pltpu.VMEM((C,), jnp.int32) for _ in range(2)],
            [pltpu.VMEM((C, H_DIM), jnp.float32) for _ in range(2)],
            [pltpu.VMEM((C, H_DIM), jnp.float32) for _ in range(2)],
            [pltpu.VMEM((C,), jnp.float32) for _ in range(2)],
            pltpu.SemaphoreType.DMA,
            [pltpu.SemaphoreType.DMA for _ in range(2)],
            [pltpu.SemaphoreType.DMA for _ in range(2)],
        ],
    )(embedding, w_relation, src, rel, dst)
    return score[:n]


# speedup vs baseline: 1.2096x; 1.0304x over previous
import functools

import jax
import jax.numpy as jnp
from jax import lax
from jax.experimental import pallas as pl
from jax.experimental.pallas import tpu as pltpu
from jax.experimental.pallas import tpu_sc as plsc

N_NODES = 100000
H_DIM = 128
NUM_RELS = 100
N_TRIPLETS = 500000

NC = 2
NS = 16
L = 16
NW = NC * NS

C = 128
FB = 16
NSLOT = 3


def _score_body(n_chunks, b_per_w,
                emb_hbm, wrel_hbm, src_hbm, rel_hbm, dst_hbm, out_hbm,
                wrel_v, sidx_v, didx_v, ridx_v, srows_v, orows_v, scores_v,
                sem_idx, sem_rows, sem_out):
    wid = lax.axis_index("s") * NC + lax.axis_index("c")
    base_w = wid * b_per_w
    pltpu.sync_copy(wrel_hbm, wrel_v)
    lane = lax.iota(jnp.int32, L)

    def start_idx(chunk, slot):
        base = base_w + chunk * C
        pltpu.async_copy(src_hbm.at[pl.ds(base, C)], sidx_v[slot], sem_idx[slot])
        pltpu.async_copy(dst_hbm.at[pl.ds(base, C)], didx_v[slot], sem_idx[slot])
        pltpu.async_copy(rel_hbm.at[pl.ds(base, C)], ridx_v[slot], sem_idx[slot])

    def wait_idx(slot):
        pltpu.make_async_copy(src_hbm.at[pl.ds(0, C)], sidx_v[slot], sem_idx[slot]).wait()
        pltpu.make_async_copy(dst_hbm.at[pl.ds(0, C)], didx_v[slot], sem_idx[slot]).wait()
        pltpu.make_async_copy(rel_hbm.at[pl.ds(0, C)], ridx_v[slot], sem_idx[slot]).wait()

    def start_rows(slot):
        pltpu.async_copy(emb_hbm.at[sidx_v[slot]], srows_v[slot], sem_rows[slot])
        pltpu.async_copy(emb_hbm.at[didx_v[slot]], orows_v[slot], sem_rows[slot])

    def wait_rows(slot):
        pltpu.make_async_copy(emb_hbm.at[sidx_v[slot]], srows_v[slot], sem_rows[slot]).wait()
        pltpu.make_async_copy(emb_hbm.at[didx_v[slot]], orows_v[slot], sem_rows[slot]).wait()

    def start_out(chunk, slot):
        pltpu.async_copy(scores_v[slot], out_hbm.at[pl.ds(base_w + chunk * C, C)],
                         sem_out[slot])

    def wait_out(slot):
        pltpu.make_async_copy(scores_v[slot], out_hbm.at[pl.ds(0, C)], sem_out[slot]).wait()

    def compute(slot):
        def group_body(g, gcarry):
            rows = lane + g * L
            rvec = ridx_v[slot][pl.ds(g * L, L)]

            def fblk_body(fb, acc):
                fbase = fb * FB
                for ff in range(FB):
                    col = jnp.full((L,), ff, jnp.int32) + fbase
                    sv = plsc.load_gather(srows_v[slot], [rows, col])
                    ov = plsc.load_gather(orows_v[slot], [rows, col])
                    rv = plsc.load_gather(wrel_v, [rvec, col])
                    acc = acc + sv * ov * rv
                return acc

            acc = lax.fori_loop(0, H_DIM // FB, fblk_body, jnp.zeros((L,), jnp.float32))
            scores_v[slot][pl.ds(g * L, L)] = acc
            return gcarry

        lax.fori_loop(0, C // L, group_body, 0)

    for s in range(NSLOT):
        start_idx(s, s)
    wait_idx(0)
    start_rows(0)
    for i in range(NSLOT):
        wait_idx((i + 1) % NSLOT)
        start_rows((i + 1) % NSLOT)
        wait_rows(i)
        compute(i)
        start_out(i, i)
        start_idx(i + NSLOT, i)

    def block_body(g, carry):
        for k in range(NSLOT):
            i = NSLOT + NSLOT * g + k
            wait_idx((k + 1) % NSLOT)
            start_rows((k + 1) % NSLOT)
            wait_rows(k)
            wait_out(k)
            compute(k)
            start_out(i, k)
            start_idx(i + NSLOT, k)
        return carry

    lax.fori_loop(0, (n_chunks - 2 * NSLOT) // NSLOT, block_body, 0)

    i = n_chunks - NSLOT
    for k in range(NSLOT - 1):
        wait_idx(k + 1)
        start_rows(k + 1)
        wait_rows(k)
        wait_out(k)
        compute(k)
        start_out(i + k, k)
    wait_rows(NSLOT - 1)
    wait_out(NSLOT - 1)
    compute(NSLOT - 1)
    start_out(n_chunks - 1, NSLOT - 1)
    for s in range(NSLOT):
        wait_out(s)


def kernel(embedding, w_relation, src, rel, dst):
    n = src.shape[0]
    step = NW * C * NSLOT
    n_pad = ((n + step - 1) // step) * step
    pad = n_pad - n
    if pad:
        zpad = jnp.zeros((pad,), src.dtype)
        src = jnp.concatenate([src, zpad])
        rel = jnp.concatenate([rel, zpad])
        dst = jnp.concatenate([dst, zpad])
    b_per_w = n_pad // NW
    n_chunks = b_per_w // C
    assert n_chunks >= 3 * NSLOT and n_chunks % NSLOT == 0

    mesh = plsc.VectorSubcoreMesh(core_axis_name="c", subcore_axis_name="s")
    body = functools.partial(_score_body, n_chunks, b_per_w)
    score = pl.kernel(
        body,
        out_type=jax.ShapeDtypeStruct((n_pad,), jnp.float32),
        mesh=mesh,
        compiler_params=pltpu.CompilerParams(needs_layout_passes=False),
        scratch_types=[
            pltpu.VMEM((NUM_RELS, H_DIM), jnp.float32),
            [pltpu.VMEM((C,), jnp.int32) for _ in range(NSLOT)],
            [pltpu.VMEM((C,), jnp.int32) for _ in range(NSLOT)],
            [pltpu.VMEM((C,), jnp.int32) for _ in range(NSLOT)],
            [pltpu.VMEM((C, H_DIM), jnp.float32) for _ in range(NSLOT)],
            [pltpu.VMEM((C, H_DIM), jnp.float32) for _ in range(NSLOT)],
            [pltpu.VMEM((C,), jnp.float32) for _ in range(NSLOT)],
            [pltpu.SemaphoreType.DMA for _ in range(NSLOT)],
            [pltpu.SemaphoreType.DMA for _ in range(NSLOT)],
            [pltpu.SemaphoreType.DMA for _ in range(NSLOT)],
        ],
    )(embedding, w_relation, src, rel, dst)
    return score[:n]


# speedup vs baseline: 3.4813x; 2.8782x over previous
import functools

import jax
import jax.numpy as jnp
from jax import lax
from jax.experimental import pallas as pl
from jax.experimental.pallas import tpu as pltpu
from jax.experimental.pallas import tpu_sc as plsc

N_NODES = 100000
H_DIM = 128
NUM_RELS = 100
N_TRIPLETS = 500000

NC = 2
NS = 16
L = 16
NW = NC * NS

C = 128
TU = 2
NSLOT = 3


def _score_body(n_chunks, b_per_w,
                emb_hbm, wrel_hbm, src_hbm, rel_hbm, dst_hbm, out_hbm,
                wrel_v, sidx_v, didx_v, ridx_v, srows_v, orows_v, scores_v,
                sem_idx, sem_rows, sem_out):
    wid = lax.axis_index("s") * NC + lax.axis_index("c")
    base_w = wid * b_per_w
    pltpu.sync_copy(wrel_hbm, wrel_v)
    lane = lax.iota(jnp.int32, L)

    def start_idx(chunk, slot):
        base = base_w + chunk * C
        pltpu.async_copy(src_hbm.at[pl.ds(base, C)], sidx_v[slot], sem_idx[slot])
        pltpu.async_copy(dst_hbm.at[pl.ds(base, C)], didx_v[slot], sem_idx[slot])
        pltpu.async_copy(rel_hbm.at[pl.ds(base, C)], ridx_v[slot].at[pl.ds(0, C)],
                         sem_idx[slot])

    def wait_idx(slot):
        pltpu.make_async_copy(src_hbm.at[pl.ds(0, C)], sidx_v[slot], sem_idx[slot]).wait()
        pltpu.make_async_copy(dst_hbm.at[pl.ds(0, C)], didx_v[slot], sem_idx[slot]).wait()
        pltpu.make_async_copy(rel_hbm.at[pl.ds(0, C)], ridx_v[slot].at[pl.ds(0, C)],
                              sem_idx[slot]).wait()

    def start_rows(slot):
        pltpu.async_copy(emb_hbm.at[sidx_v[slot]], srows_v[slot], sem_rows[slot])
        pltpu.async_copy(emb_hbm.at[didx_v[slot]], orows_v[slot], sem_rows[slot])

    def wait_rows(slot):
        pltpu.make_async_copy(emb_hbm.at[sidx_v[slot]], srows_v[slot], sem_rows[slot]).wait()
        pltpu.make_async_copy(emb_hbm.at[didx_v[slot]], orows_v[slot], sem_rows[slot]).wait()

    def start_out(chunk, slot):
        pltpu.async_copy(scores_v[slot], out_hbm.at[pl.ds(base_w + chunk * C, C)],
                         sem_out[slot])

    def wait_out(slot):
        pltpu.make_async_copy(scores_v[slot], out_hbm.at[pl.ds(0, C)], sem_out[slot]).wait()

    def compute(slot):
        zero = jnp.zeros((L,), jnp.float32)

        def zero_body(g, gcarry):
            scores_v[slot][pl.ds(g * L, L)] = zero
            return gcarry

        lax.fori_loop(0, C // L, zero_body, 0)

        def trip_body(tt, tcarry):
            rvec = ridx_v[slot][pl.ds(tt * TU, L)] * H_DIM
            for u in range(TU):
                t = tt * TU + u
                rbase = rvec[u]
                acc0 = zero
                acc1 = zero
                for k in range(H_DIM // L):
                    sv = srows_v[slot][t, pl.ds(k * L, L)]
                    ov = orows_v[slot][t, pl.ds(k * L, L)]
                    rv = wrel_v[pl.ds(rbase + k * L, L)]
                    prod = sv * ov * rv
                    if k % 2 == 0:
                        acc0 = acc0 + prod
                    else:
                        acc1 = acc1 + prod
                plsc.addupdate_scatter(scores_v[slot],
                                       [jnp.full((L,), t, jnp.int32)],
                                       acc0 + acc1)
            return tcarry

        lax.fori_loop(0, C // TU, trip_body, 0)

    for s in range(NSLOT):
        start_idx(s, s)
    wait_idx(0)
    start_rows(0)
    for i in range(NSLOT):
        wait_idx((i + 1) % NSLOT)
        start_rows((i + 1) % NSLOT)
        wait_rows(i)
        compute(i)
        start_out(i, i)
        start_idx(i + NSLOT, i)

    def block_body(g, carry):
        for k in range(NSLOT):
            i = NSLOT + NSLOT * g + k
            wait_idx((k + 1) % NSLOT)
            start_rows((k + 1) % NSLOT)
            wait_rows(k)
            wait_out(k)
            compute(k)
            start_out(i, k)
            start_idx(i + NSLOT, k)
        return carry

    lax.fori_loop(0, (n_chunks - 2 * NSLOT) // NSLOT, block_body, 0)

    i = n_chunks - NSLOT
    for k in range(NSLOT - 1):
        wait_idx(k + 1)
        start_rows(k + 1)
        wait_rows(k)
        wait_out(k)
        compute(k)
        start_out(i + k, k)
    wait_rows(NSLOT - 1)
    wait_out(NSLOT - 1)
    compute(NSLOT - 1)
    start_out(n_chunks - 1, NSLOT - 1)
    for s in range(NSLOT):
        wait_out(s)


def kernel(embedding, w_relation, src, rel, dst):
    n = src.shape[0]
    step = NW * C * NSLOT
    n_pad = ((n + step - 1) // step) * step
    pad = n_pad - n
    if pad:
        zpad = jnp.zeros((pad,), src.dtype)
        src = jnp.concatenate([src, zpad])
        rel = jnp.concatenate([rel, zpad])
        dst = jnp.concatenate([dst, zpad])
    b_per_w = n_pad // NW
    n_chunks = b_per_w // C
    assert n_chunks >= 3 * NSLOT and n_chunks % NSLOT == 0

    mesh = plsc.VectorSubcoreMesh(core_axis_name="c", subcore_axis_name="s")
    body = functools.partial(_score_body, n_chunks, b_per_w)
    score = pl.kernel(
        body,
        out_type=jax.ShapeDtypeStruct((n_pad,), jnp.float32),
        mesh=mesh,
        compiler_params=pltpu.CompilerParams(needs_layout_passes=False),
        scratch_types=[
            pltpu.VMEM((NUM_RELS * H_DIM,), jnp.float32),
            [pltpu.VMEM((C,), jnp.int32) for _ in range(NSLOT)],
            [pltpu.VMEM((C,), jnp.int32) for _ in range(NSLOT)],
            [pltpu.VMEM((C + L,), jnp.int32) for _ in range(NSLOT)],
            [pltpu.VMEM((C, H_DIM), jnp.float32) for _ in range(NSLOT)],
            [pltpu.VMEM((C, H_DIM), jnp.float32) for _ in range(NSLOT)],
            [pltpu.VMEM((C,), jnp.float32) for _ in range(NSLOT)],
            [pltpu.SemaphoreType.DMA for _ in range(NSLOT)],
            [pltpu.SemaphoreType.DMA for _ in range(NSLOT)],
            [pltpu.SemaphoreType.DMA for _ in range(NSLOT)],
        ],
    )(embedding, w_relation.reshape(-1), src, rel, dst)
    return score[:n]
